# baseline (speedup 1.0000x reference)
import functools

import jax
import jax.numpy as jnp
from jax import lax
from jax.experimental import pallas as pl
from jax.experimental.pallas import tpu as pltpu
from jax.experimental.pallas import tpu_sc as plsc

N = 10000
E = 320000
D = 128

NC = 2
NS = 16
NW = NC * NS
E_PER_W = E // NW
B = 80
STEPS = E_PER_W // B
NP = 10240
R_PER_T = NP // NS
RB = 80


def _msg_body(x_ref, wt_ref, m_ref):
    e = jnp.dot(x_ref[...], wt_ref[...], preferred_element_type=jnp.float32)
    g = jax.nn.sigmoid(e[:, :D])
    m_ref[...] = g * e[:, D:]


def _msg(x, wt):
    blk = 1000
    return pl.pallas_call(
        _msg_body,
        grid=(N // blk,),
        in_specs=[
            pl.BlockSpec((blk, D), lambda i: (i, 0)),
            pl.BlockSpec((D, 2 * D), lambda i: (0, 0)),
        ],
        out_specs=pl.BlockSpec((blk, D), lambda i: (i, 0)),
        out_shape=jax.ShapeDtypeStruct((N, D), jnp.float32),
    )(x, wt)


def _make_scatter():
    mesh = plsc.VectorSubcoreMesh(core_axis_name="c", subcore_axis_name="s")

    @functools.partial(
        pl.kernel,
        out_type=jax.ShapeDtypeStruct((NC, NP, D), jnp.float32),
        mesh=mesh,
        scratch_types=[
            pltpu.VMEM((E_PER_W,), jnp.int32),
            pltpu.VMEM((E_PER_W,), jnp.int32),
            pltpu.VMEM((B, D), jnp.float32),
            pltpu.VMEM((B, D), jnp.float32),
            pltpu.VMEM_SHARED((NP, D), jnp.float32),
            pltpu.SemaphoreType.DMA,
            pltpu.SemaphoreType.DMA,
            pltpu.SemaphoreType.DMA,
            pltpu.SemaphoreType.DMA,
        ],
    )
    def scatter_k(m_hbm, src_hbm, tgt_hbm, init_hbm, out_hbm,
                  tgt_v, src_v, rows0, rows1, acc,
                  semg0, semg1, sems0, sems1):
        c = lax.axis_index("c")
        s = lax.axis_index("s")
        wid = s * NC + c

        pltpu.sync_copy(tgt_hbm.at[wid], tgt_v)
        pltpu.sync_copy(src_hbm.at[wid], src_v)

        rbase = s * R_PER_T

        def init_step(j, carry):
            r = rbase + j * 2 * RB
            pltpu.sync_copy(init_hbm.at[pl.ds(r, RB)], rows0.at[pl.ds(0, RB)])
            pltpu.sync_copy(rows0.at[pl.ds(0, RB)], acc.at[pl.ds(r, RB)])
            pltpu.sync_copy(init_hbm.at[pl.ds(r + RB, RB)], rows1.at[pl.ds(0, RB)])
            pltpu.sync_copy(rows1.at[pl.ds(0, RB)], acc.at[pl.ds(r + RB, RB)])
            return carry

        lax.fori_loop(0, R_PER_T // (2 * RB), init_step, 0)
        plsc.subcore_barrier()

        def wait_g(buf, sem):
            pltpu.make_async_copy(m_hbm.at[pl.ds(0, B)], buf, sem).wait()

        def wait_s(buf, sem):
            pltpu.make_async_copy(buf, acc.at[pl.ds(0, B)], sem).wait()

        def tslice(ref, i):
            return ref.at[pl.ds(i * B, B)]

        pltpu.async_copy(m_hbm.at[tslice(tgt_v, 0)], rows0, semg0)
        pltpu.async_copy(m_hbm.at[tslice(tgt_v, 1)], rows1, semg1)

        def pair(k, carry):
            i = 2 * k
            wait_g(rows0, semg0)
            pltpu.async_copy(rows0, acc.at[tslice(src_v, i)], sems0, add=True)
            wait_g(rows1, semg1)
            pltpu.async_copy(rows1, acc.at[tslice(src_v, i + 1)], sems1, add=True)
            wait_s(rows0, sems0)

            @pl.when(i + 2 < STEPS)
            def _():
                pltpu.async_copy(m_hbm.at[tslice(tgt_v, i + 2)], rows0, semg0)

            wait_s(rows1, sems1)

            @pl.when(i + 3 < STEPS)
            def _():
                pltpu.async_copy(m_hbm.at[tslice(tgt_v, i + 3)], rows1, semg1)

            return carry

        lax.fori_loop(0, STEPS // 2, pair, 0)
        wait_g(rows0, semg0)
        pltpu.sync_copy(rows0, acc.at[tslice(src_v, STEPS - 1)], add=True)
        plsc.subcore_barrier()

        def out_step(j, carry):
            r = rbase + j * 2 * RB
            pltpu.sync_copy(acc.at[pl.ds(r, RB)], rows0.at[pl.ds(0, RB)])
            pltpu.sync_copy(rows0.at[pl.ds(0, RB)], out_hbm.at[c, pl.ds(r, RB)])
            pltpu.sync_copy(acc.at[pl.ds(r + RB, RB)], rows1.at[pl.ds(0, RB)])
            pltpu.sync_copy(rows1.at[pl.ds(0, RB)], out_hbm.at[c, pl.ds(r + RB, RB)])
            return carry

        lax.fori_loop(0, R_PER_T // (2 * RB), out_step, 0)

    return scatter_k


_scatter_k = _make_scatter()


def _add_body(x_ref, p_ref, o_ref):
    o_ref[...] = x_ref[...] + p_ref[0] + p_ref[1]


def _combine(x, p):
    blk = 1000
    return pl.pallas_call(
        _add_body,
        grid=(N // blk,),
        in_specs=[
            pl.BlockSpec((blk, D), lambda i: (i, 0)),
            pl.BlockSpec((NC, blk, D), lambda i: (0, i, 0)),
        ],
        out_specs=pl.BlockSpec((blk, D), lambda i: (i, 0)),
        out_shape=jax.ShapeDtypeStruct((N, D), jnp.float32),
    )(x, p)


def kernel(input, edge_sources, edge_targets, distance_nbr, W):
    x = input
    m = _msg(x, W.T)
    src = edge_sources.astype(jnp.int32).reshape(NW, E_PER_W)
    tgt = edge_targets.astype(jnp.int32).reshape(NW, E_PER_W)
    init = jnp.zeros((NP, D), jnp.float32)
    p = _scatter_k(m, src, tgt, init)
    return _combine(x, p)


# speedup vs baseline: 1.2067x; 1.2067x over previous
import functools

import jax
import jax.numpy as jnp
from jax import lax
from jax.experimental import pallas as pl
from jax.experimental.pallas import tpu as pltpu
from jax.experimental.pallas import tpu_sc as plsc

N = 10000
E = 320000
D = 128

NC = 2
NS = 16
NW = NC * NS
E_PER_W = E // NW
B = 80
STEPS = E_PER_W // B
NP = 10240
R_PER_T = NP // NS
RB = 80


def _msg_body(x_ref, wt_ref, m_ref):
    e = jnp.dot(x_ref[...], wt_ref[...], preferred_element_type=jnp.float32)
    g = jax.nn.sigmoid(e[:, :D])
    m_ref[...] = g * e[:, D:]


def _msg(x, wt):
    blk = 1000
    return pl.pallas_call(
        _msg_body,
        grid=(N // blk,),
        in_specs=[
            pl.BlockSpec((blk, D), lambda i: (i, 0)),
            pl.BlockSpec((D, 2 * D), lambda i: (0, 0)),
        ],
        out_specs=pl.BlockSpec((blk, D), lambda i: (i, 0)),
        out_shape=jax.ShapeDtypeStruct((N, D), jnp.float32),
    )(x, wt)


def _make_scatter():
    mesh = plsc.VectorSubcoreMesh(core_axis_name="c", subcore_axis_name="s")

    @functools.partial(
        pl.kernel,
        out_type=jax.ShapeDtypeStruct((NC, NP, D), jnp.float32),
        mesh=mesh,
        scratch_types=[
            pltpu.VMEM((E_PER_W,), jnp.int32),
            pltpu.VMEM((E_PER_W,), jnp.int32),
            pltpu.VMEM((B, D), jnp.float32),
            pltpu.VMEM((B, D), jnp.float32),
            pltpu.VMEM_SHARED((NP, D), jnp.float32),
            pltpu.SemaphoreType.DMA,
            pltpu.SemaphoreType.DMA,
            pltpu.SemaphoreType.DMA,
            pltpu.SemaphoreType.DMA,
        ],
    )
    def scatter_k(m_hbm, src_hbm, tgt_hbm, init_hbm, out_hbm,
                  tgt_v, src_v, rows0, rows1, acc,
                  semg0, semg1, sems0, sems1):
        c = lax.axis_index("c")
        s = lax.axis_index("s")
        wid = s * NC + c

        pltpu.sync_copy(tgt_hbm.at[wid], tgt_v)
        pltpu.sync_copy(src_hbm.at[wid], src_v)

        rbase = s * R_PER_T

        def init_step(j, carry):
            r = rbase + j * 2 * RB
            pltpu.sync_copy(init_hbm.at[pl.ds(r, RB)], rows0.at[pl.ds(0, RB)])
            pltpu.sync_copy(rows0.at[pl.ds(0, RB)], acc.at[pl.ds(r, RB)])
            pltpu.sync_copy(init_hbm.at[pl.ds(r + RB, RB)], rows1.at[pl.ds(0, RB)])
            pltpu.sync_copy(rows1.at[pl.ds(0, RB)], acc.at[pl.ds(r + RB, RB)])
            return carry

        lax.fori_loop(0, R_PER_T // (2 * RB), init_step, 0)
        plsc.subcore_barrier()

        def wait_g(buf, sem):
            pltpu.make_async_copy(m_hbm.at[pl.ds(0, B)], buf, sem).wait()

        def wait_s(buf, sem):
            pltpu.make_async_copy(buf, acc.at[pl.ds(0, B)], sem).wait()

        def tslice(ref, i):
            return ref.at[pl.ds(i * B, B)]

        pltpu.async_copy(m_hbm.at[tslice(tgt_v, 0)], rows0, semg0)

        def pair(k, carry):
            i = 2 * k
            pltpu.async_copy(m_hbm.at[tslice(tgt_v, i + 1)], rows1, semg1)
            wait_g(rows0, semg0)
            pltpu.sync_copy(rows0, acc.at[tslice(src_v, i)], add=True)

            @pl.when(i + 2 < STEPS)
            def _():
                pltpu.async_copy(m_hbm.at[tslice(tgt_v, i + 2)], rows0, semg0)

            wait_g(rows1, semg1)
            pltpu.sync_copy(rows1, acc.at[tslice(src_v, i + 1)], add=True)
            return carry

        lax.fori_loop(0, STEPS // 2, pair, 0)
        wait_g(rows0, semg0)
        pltpu.sync_copy(rows0, acc.at[tslice(src_v, STEPS - 1)], add=True)
        plsc.subcore_barrier()

        def out_step(j, carry):
            r = rbase + j * 2 * RB
            pltpu.sync_copy(acc.at[pl.ds(r, RB)], rows0.at[pl.ds(0, RB)])
            pltpu.sync_copy(rows0.at[pl.ds(0, RB)], out_hbm.at[c, pl.ds(r, RB)])
            pltpu.sync_copy(acc.at[pl.ds(r + RB, RB)], rows1.at[pl.ds(0, RB)])
            pltpu.sync_copy(rows1.at[pl.ds(0, RB)], out_hbm.at[c, pl.ds(r + RB, RB)])
            return carry

        lax.fori_loop(0, R_PER_T // (2 * RB), out_step, 0)

    return scatter_k


_scatter_k = _make_scatter()


def _add_body(x_ref, p_ref, o_ref):
    o_ref[...] = x_ref[...] + p_ref[0] + p_ref[1]


def _combine(x, p):
    blk = 1000
    return pl.pallas_call(
        _add_body,
        grid=(N // blk,),
        in_specs=[
            pl.BlockSpec((blk, D), lambda i: (i, 0)),
            pl.BlockSpec((NC, blk, D), lambda i: (0, i, 0)),
        ],
        out_specs=pl.BlockSpec((blk, D), lambda i: (i, 0)),
        out_shape=jax.ShapeDtypeStruct((N, D), jnp.float32),
    )(x, p)


def kernel(input, edge_sources, edge_targets, distance_nbr, W):
    x = input
    m = _msg(x, W.T)
    src = edge_sources.astype(jnp.int32).reshape(NW, E_PER_W)
    tgt = edge_targets.astype(jnp.int32).reshape(NW, E_PER_W)
    init = jnp.zeros((NP, D), jnp.float32)
    p = _scatter_k(m, src, tgt, init)
    return _combine(x, p)


# speedup vs baseline: 1.2637x; 1.0472x over previous
import functools

import jax
import jax.numpy as jnp
from jax import lax
from jax.experimental import pallas as pl
from jax.experimental.pallas import tpu as pltpu
from jax.experimental.pallas import tpu_sc as plsc

N = 10000
E = 320000
D = 128

NC = 2
NS = 16
NW = NC * NS
E_PER_W = E // NW
B = 104
FULL_STEPS = 96
BT = E_PER_W - FULL_STEPS * B
NP = 10240
R_PER_T = NP // NS
RB = 80


def _msg_body(x_ref, wt_ref, m_ref):
    e = jnp.dot(x_ref[...], wt_ref[...], preferred_element_type=jnp.float32)
    g = jax.nn.sigmoid(e[:, :D])
    m_ref[...] = g * e[:, D:]


def _msg(x, wt):
    blk = 1000
    return pl.pallas_call(
        _msg_body,
        grid=(N // blk,),
        in_specs=[
            pl.BlockSpec((blk, D), lambda i: (i, 0)),
            pl.BlockSpec((D, 2 * D), lambda i: (0, 0)),
        ],
        out_specs=pl.BlockSpec((blk, D), lambda i: (i, 0)),
        out_shape=jax.ShapeDtypeStruct((N, D), jnp.float32),
    )(x, wt)


def _make_scatter():
    mesh = plsc.VectorSubcoreMesh(core_axis_name="c", subcore_axis_name="s")

    @functools.partial(
        pl.kernel,
        out_type=jax.ShapeDtypeStruct((NC, NP, D), jnp.float32),
        mesh=mesh,
        scratch_types=[
            pltpu.VMEM((E_PER_W,), jnp.int32),
            pltpu.VMEM((E_PER_W,), jnp.int32),
            pltpu.VMEM((B, D), jnp.float32),
            pltpu.VMEM((B, D), jnp.float32),
            pltpu.VMEM_SHARED((NP, D), jnp.float32),
            pltpu.SemaphoreType.DMA,
            pltpu.SemaphoreType.DMA,
            pltpu.SemaphoreType.DMA,
            pltpu.SemaphoreType.DMA,
        ],
    )
    def scatter_k(m_hbm, src_hbm, tgt_hbm, init_hbm, out_hbm,
                  tgt_v, src_v, rows0, rows1, acc,
                  semg0, semg1, sems0, sems1):
        c = lax.axis_index("c")
        s = lax.axis_index("s")
        wid = s * NC + c

        pltpu.sync_copy(tgt_hbm.at[wid], tgt_v)
        pltpu.sync_copy(src_hbm.at[wid], src_v)

        rbase = s * R_PER_T

        def init_step(j, carry):
            r = rbase + j * 2 * RB
            pltpu.sync_copy(init_hbm.at[pl.ds(r, RB)], rows0.at[pl.ds(0, RB)])
            pltpu.sync_copy(rows0.at[pl.ds(0, RB)], acc.at[pl.ds(r, RB)])
            pltpu.sync_copy(init_hbm.at[pl.ds(r + RB, RB)], rows1.at[pl.ds(0, RB)])
            pltpu.sync_copy(rows1.at[pl.ds(0, RB)], acc.at[pl.ds(r + RB, RB)])
            return carry

        lax.fori_loop(0, R_PER_T // (2 * RB), init_step, 0)
        plsc.subcore_barrier()

        def wait_g(buf, sem):
            pltpu.make_async_copy(m_hbm.at[pl.ds(0, B)], buf, sem).wait()

        def wait_s(buf, sem):
            pltpu.make_async_copy(buf, acc.at[pl.ds(0, B)], sem).wait()

        def tslice(ref, i):
            return ref.at[pl.ds(i * B, B)]

        pltpu.async_copy(m_hbm.at[tslice(tgt_v, 0)], rows0, semg0)

        def pair(k, carry):
            i = 2 * k
            pltpu.async_copy(m_hbm.at[tslice(tgt_v, i + 1)], rows1, semg1)
            wait_g(rows0, semg0)
            pltpu.sync_copy(rows0, acc.at[tslice(src_v, i)], add=True)

            @pl.when(i + 2 < FULL_STEPS)
            def _():
                pltpu.async_copy(m_hbm.at[tslice(tgt_v, i + 2)], rows0, semg0)

            wait_g(rows1, semg1)
            pltpu.sync_copy(rows1, acc.at[tslice(src_v, i + 1)], add=True)
            return carry

        lax.fori_loop(0, FULL_STEPS // 2, pair, 0)
        tb = FULL_STEPS * B
        pltpu.async_copy(
            m_hbm.at[tgt_v.at[pl.ds(tb, BT)]], rows0.at[pl.ds(0, BT)], semg0)
        pltpu.make_async_copy(
            m_hbm.at[pl.ds(0, BT)], rows0.at[pl.ds(0, BT)], semg0).wait()
        pltpu.sync_copy(
            rows0.at[pl.ds(0, BT)], acc.at[src_v.at[pl.ds(tb, BT)]], add=True)
        plsc.subcore_barrier()

        def out_step(j, carry):
            r = rbase + j * 2 * RB
            pltpu.sync_copy(acc.at[pl.ds(r, RB)], rows0.at[pl.ds(0, RB)])
            pltpu.sync_copy(rows0.at[pl.ds(0, RB)], out_hbm.at[c, pl.ds(r, RB)])
            pltpu.sync_copy(acc.at[pl.ds(r + RB, RB)], rows1.at[pl.ds(0, RB)])
            pltpu.sync_copy(rows1.at[pl.ds(0, RB)], out_hbm.at[c, pl.ds(r + RB, RB)])
            return carry

        lax.fori_loop(0, R_PER_T // (2 * RB), out_step, 0)

    return scatter_k


_scatter_k = _make_scatter()


def _add_body(x_ref, p_ref, o_ref):
    o_ref[...] = x_ref[...] + p_ref[0] + p_ref[1]


def _combine(x, p):
    blk = 1000
    return pl.pallas_call(
        _add_body,
        grid=(N // blk,),
        in_specs=[
            pl.BlockSpec((blk, D), lambda i: (i, 0)),
            pl.BlockSpec((NC, blk, D), lambda i: (0, i, 0)),
        ],
        out_specs=pl.BlockSpec((blk, D), lambda i: (i, 0)),
        out_shape=jax.ShapeDtypeStruct((N, D), jnp.float32),
    )(x, p)


def kernel(input, edge_sources, edge_targets, distance_nbr, W):
    x = input
    m = _msg(x, W.T)
    src = edge_sources.astype(jnp.int32).reshape(NW, E_PER_W)
    tgt = edge_targets.astype(jnp.int32).reshape(NW, E_PER_W)
    init = jnp.zeros((NP, D), jnp.float32)
    p = _scatter_k(m, src, tgt, init)
    return _combine(x, p)


# speedup vs baseline: 1.3293x; 1.0519x over previous
import functools

import jax
import jax.numpy as jnp
from jax import lax
from jax.experimental import pallas as pl
from jax.experimental.pallas import tpu as pltpu
from jax.experimental.pallas import tpu_sc as plsc

N = 10000
E = 320000
D = 128

NC = 2
NS = 16
NW = NC * NS
E_PER_W = E // NW
B = 104
FULL_STEPS = 96
BT = E_PER_W - FULL_STEPS * B
NP = 10240
R_PER_T = NP // NS
RB = 80


def _msg_body(x_ref, wt_ref, m_ref):
    e = jnp.dot(x_ref[...], wt_ref[...], preferred_element_type=jnp.float32)
    g = jax.nn.sigmoid(e[:, :D])
    m_ref[...] = g * e[:, D:]


def _msg(x, wt):
    blk = 1000
    return pl.pallas_call(
        _msg_body,
        grid=(N // blk,),
        in_specs=[
            pl.BlockSpec((blk, D), lambda i: (i, 0)),
            pl.BlockSpec((D, 2 * D), lambda i: (0, 0)),
        ],
        out_specs=pl.BlockSpec((blk, D), lambda i: (i, 0)),
        out_shape=jax.ShapeDtypeStruct((N, D), jnp.float32),
    )(x, wt)


def _make_scatter():
    mesh = plsc.VectorSubcoreMesh(core_axis_name="c", subcore_axis_name="s")

    @functools.partial(
        pl.kernel,
        out_type=jax.ShapeDtypeStruct((NC, NP, D), jnp.float32),
        mesh=mesh,
        scratch_types=[
            pltpu.VMEM((E_PER_W,), jnp.int32),
            pltpu.VMEM((E_PER_W,), jnp.int32),
            pltpu.VMEM((B, D), jnp.float32),
            pltpu.VMEM((B, D), jnp.float32),
            pltpu.VMEM_SHARED((NP, D), jnp.float32),
            pltpu.SemaphoreType.DMA,
            pltpu.SemaphoreType.DMA,
            pltpu.SemaphoreType.DMA,
            pltpu.SemaphoreType.DMA,
        ],
    )
    def scatter_k(m_hbm, src_hbm, tgt_hbm, init_hbm, out_hbm,
                  tgt_v, src_v, rows0, rows1, acc,
                  semg0, semg1, sems0, sems1):
        c = lax.axis_index("c")
        s = lax.axis_index("s")
        wid = s * NC + c

        cp_t = pltpu.async_copy(tgt_hbm.at[wid], tgt_v, semg0)
        cp_s = pltpu.async_copy(src_hbm.at[wid], src_v, semg1)

        rbase = s * R_PER_T
        pltpu.sync_copy(init_hbm.at[pl.ds(rbase, R_PER_T)],
                        acc.at[pl.ds(rbase, R_PER_T)])
        cp_t.wait()
        cp_s.wait()
        plsc.subcore_barrier()

        def wait_g(buf, sem):
            pltpu.make_async_copy(m_hbm.at[pl.ds(0, B)], buf, sem).wait()

        def wait_s(buf, sem):
            pltpu.make_async_copy(buf, acc.at[pl.ds(0, B)], sem).wait()

        def tslice(ref, i):
            return ref.at[pl.ds(i * B, B)]

        pltpu.async_copy(m_hbm.at[tslice(tgt_v, 0)], rows0, semg0)

        def pair(k, carry):
            i = 2 * k
            pltpu.async_copy(m_hbm.at[tslice(tgt_v, i + 1)], rows1, semg1)
            wait_g(rows0, semg0)
            pltpu.sync_copy(rows0, acc.at[tslice(src_v, i)], add=True)

            @pl.when(i + 2 < FULL_STEPS)
            def _():
                pltpu.async_copy(m_hbm.at[tslice(tgt_v, i + 2)], rows0, semg0)

            wait_g(rows1, semg1)
            pltpu.sync_copy(rows1, acc.at[tslice(src_v, i + 1)], add=True)
            return carry

        lax.fori_loop(0, FULL_STEPS // 2, pair, 0)
        tb = FULL_STEPS * B
        pltpu.async_copy(
            m_hbm.at[tgt_v.at[pl.ds(tb, BT)]], rows0.at[pl.ds(0, BT)], semg0)
        pltpu.make_async_copy(
            m_hbm.at[pl.ds(0, BT)], rows0.at[pl.ds(0, BT)], semg0).wait()
        pltpu.sync_copy(
            rows0.at[pl.ds(0, BT)], acc.at[src_v.at[pl.ds(tb, BT)]], add=True)
        plsc.subcore_barrier()

        pltpu.sync_copy(acc.at[pl.ds(rbase, R_PER_T)],
                        out_hbm.at[c, pl.ds(rbase, R_PER_T)])

    return scatter_k


_scatter_k = _make_scatter()


def _add_body(x_ref, p_ref, o_ref):
    o_ref[...] = x_ref[...] + p_ref[0] + p_ref[1]


def _combine(x, p):
    blk = 1000
    return pl.pallas_call(
        _add_body,
        grid=(N // blk,),
        in_specs=[
            pl.BlockSpec((blk, D), lambda i: (i, 0)),
            pl.BlockSpec((NC, blk, D), lambda i: (0, i, 0)),
        ],
        out_specs=pl.BlockSpec((blk, D), lambda i: (i, 0)),
        out_shape=jax.ShapeDtypeStruct((N, D), jnp.float32),
    )(x, p)


def kernel(input, edge_sources, edge_targets, distance_nbr, W):
    x = input
    m = _msg(x, W.T)
    src = edge_sources.astype(jnp.int32).reshape(NW, E_PER_W)
    tgt = edge_targets.astype(jnp.int32).reshape(NW, E_PER_W)
    init = jnp.zeros((NP, D), jnp.float32)
    p = _scatter_k(m, src, tgt, init)
    return _combine(x, p)
